# baseline (speedup 1.0000x reference)
import jax
import jax.numpy as jnp
from jax import lax
from jax.experimental import pallas as pl
from jax.experimental.pallas import tpu as pltpu
from jax.experimental.pallas import tpu_sc as plsc

B, L, H = 4096, 200, 64
N = B * L
NC, NS = 2, 16
NW = NC * NS
RPW = B // NW
CHUNK = L
FW = 8
NROWS8 = 1000000 // FW
PER_W = RPW * L


def _sc_gather_body(rpw, idx_hbm, id_tab, f0_tab, f1_tab, id_out, fm_out,
                    idx_v, idxc0, idx80, rowsa0, rowsb0, d00, d10, fmask0,
                    idxc1, idx81, rowsa1, rowsb1, d01, d11, fmask1,
                    semi0, semf0, semi1, semf1):
    per_w = rpw * L
    wid = lax.axis_index("s") * NC + lax.axis_index("c")
    pltpu.sync_copy(idx_hbm.at[wid], idx_v)

    iota = lax.iota(jnp.int32, 16)
    tv_off = (iota >> 1) & 1
    evenl = (iota & 1) == 0
    low4 = iota < 4
    WINS = [0, 16, 32, 48, 64, 80, 96, 112, 128, 144, 160, 176, 184]
    bufs = ((idxc0, idx80, rowsa0, rowsb0, d00, d10, fmask0, semi0, semf0),
            (idxc1, idx81, rowsa1, rowsb1, d01, d11, fmask1, semi1, semf1))

    def fire(j, buf):
        idxc_v, idx8_v, rows_a, rows_b, d0_v, d1_v, _, sem_id, sem_fr = buf
        for off in WINS:
            v = idx_v[j, pl.ds(off, 16)]
            idxc_v[pl.ds(off, 16)] = v
            idx8_v[pl.ds(off, 16)] = v >> 3
        pltpu.async_copy(id_tab.at[idxc_v.at[pl.ds(0, 128)]], rows_a, sem_id)
        pltpu.async_copy(id_tab.at[idxc_v.at[pl.ds(128, 72)]], rows_b, sem_id)
        pltpu.async_copy(f0_tab.at[idx8_v.at[pl.ds(0, 128)]],
                         d0_v.at[pl.ds(0, 128)], sem_fr)
        pltpu.async_copy(f0_tab.at[idx8_v.at[pl.ds(128, 72)]],
                         d0_v.at[pl.ds(128, 72)], sem_fr)
        pltpu.async_copy(f1_tab.at[idx8_v.at[pl.ds(0, 128)]],
                         d1_v.at[pl.ds(0, 128)], sem_fr)
        pltpu.async_copy(f1_tab.at[idx8_v.at[pl.ds(128, 72)]],
                         d1_v.at[pl.ds(128, 72)], sem_fr)

    def consume(j, buf):
        idxc_v, idx8_v, rows_a, rows_b, d0_v, d1_v, fmask_v, sem_id, sem_fr = buf
        base = wid * per_w + j * CHUNK
        pltpu.make_async_copy(id_tab.at[idxc_v.at[pl.ds(0, 128)]], rows_a,
                              sem_id).wait()
        pltpu.make_async_copy(id_tab.at[idxc_v.at[pl.ds(128, 72)]], rows_b,
                              sem_id).wait()
        pltpu.sync_copy(rows_a, id_out.at[pl.ds(base, 128)])
        pltpu.sync_copy(rows_b, id_out.at[pl.ds(base + 128, 72)])
        pltpu.make_async_copy(f0_tab.at[idx8_v.at[pl.ds(0, 128)]],
                              d0_v.at[pl.ds(0, 128)], sem_fr).wait()
        pltpu.make_async_copy(f0_tab.at[idx8_v.at[pl.ds(128, 72)]],
                              d0_v.at[pl.ds(128, 72)], sem_fr).wait()
        pltpu.make_async_copy(f1_tab.at[idx8_v.at[pl.ds(0, 128)]],
                              d1_v.at[pl.ds(0, 128)], sem_fr).wait()
        pltpu.make_async_copy(f1_tab.at[idx8_v.at[pl.ds(128, 72)]],
                              d1_v.at[pl.ds(128, 72)], sem_fr).wait()
        for i in range(100):
            tv = tv_off + 2 * i
            m = plsc.load_gather(idxc_v, [tv]) & 7
            a = plsc.load_gather(d0_v, [tv, m])
            bb = plsc.load_gather(d1_v, [tv, m])
            v = jnp.where(evenl, a, bb)
            fmask_v[i] = jnp.where(low4, v, 0.0)
        rr0 = base // 2
        row0 = 200 * (rr0 // 1600) + rr0 % 200
        km = (rr0 % 1600) // 200
        pltpu.sync_copy(fmask_v, fm_out.at[pl.ds(row0, 100), pl.ds(16 * km, 16)])

    fire(0, bufs[0])

    def body(jj, carry):
        j0 = 2 * jj
        fire(j0 + 1, bufs[1])
        consume(j0, bufs[0])

        @pl.when(jj < rpw // 2 - 1)
        def _():
            fire(j0 + 2, bufs[0])

        consume(j0 + 1, bufs[1])
        return carry

    lax.fori_loop(0, rpw // 2, body, 0)


_SC_CACHE = {}


def _sc_gather_call(rpw):
    if rpw not in _SC_CACHE:
        import functools
        nseg = rpw * NW * L
        _SC_CACHE[rpw] = pl.kernel(
            functools.partial(_sc_gather_body, rpw),
            out_type=(jax.ShapeDtypeStruct((nseg, H), jnp.float32),
                      jax.ShapeDtypeStruct((nseg // 16, 128), jnp.float32)),
            mesh=plsc.VectorSubcoreMesh(core_axis_name="c", subcore_axis_name="s"),
            compiler_params=pltpu.CompilerParams(use_tc_tiling_on_sc=False,
                                                 needs_layout_passes=False),
            scratch_types=[
                pltpu.VMEM((rpw, L), jnp.int32),
                pltpu.VMEM((L,), jnp.int32),
                pltpu.VMEM((L,), jnp.int32),
                pltpu.VMEM((128, H), jnp.float32),
                pltpu.VMEM((72, H), jnp.float32),
                pltpu.VMEM((L, FW), jnp.float32),
                pltpu.VMEM((L, FW), jnp.float32),
                pltpu.VMEM((100, 16), jnp.float32),
                pltpu.VMEM((L,), jnp.int32),
                pltpu.VMEM((L,), jnp.int32),
                pltpu.VMEM((128, H), jnp.float32),
                pltpu.VMEM((72, H), jnp.float32),
                pltpu.VMEM((L, FW), jnp.float32),
                pltpu.VMEM((L, FW), jnp.float32),
                pltpu.VMEM((100, 16), jnp.float32),
                pltpu.SemaphoreType.DMA,
                pltpu.SemaphoreType.DMA,
                pltpu.SemaphoreType.DMA,
                pltpu.SemaphoreType.DMA,
            ],
        )
    return _SC_CACHE[rpw]


BT = 3200
BTH = BT // 2
BR = 16


def _tc_body(id_ref, fm_ref, w1a_ref, w1b_ref, b1_ref, bd_ref, b2_ref,
             m_ref, g_ref, be_ref, out_ref):
    fm = fm_ref[...]
    w1a = w1a_ref[...]
    w1b = w1b_ref[...]
    b1d = b1_ref[...]
    bd = bd_ref[...]
    b2d = b2_ref[...]
    mm = m_ref[...]
    g2 = g_ref[...]
    be2 = be_ref[...]
    lmask = lax.broadcasted_iota(jnp.int32, (1, 128), 1) < H
    for k in range(8):
        c = 16 * k
        F0 = jnp.where(lmask, fm[:, c + 0:c + 1], fm[:, c + 2:c + 3])
        F1 = jnp.where(lmask, fm[:, c + 1:c + 2], fm[:, c + 3:c + 4])
        h = jnp.tanh(F0 * w1a + F1 * w1b + b1d)
        fe = jnp.dot(h, bd, preferred_element_type=jnp.float32)
        x = id_ref[pl.ds(L * k, L), :] + fe + b2d
        mu = jnp.dot(x, mm, preferred_element_type=jnp.float32)
        d = x - mu
        var = jnp.dot(d * d, mm, preferred_element_type=jnp.float32)
        out_ref[pl.ds(L * k, L), :] = d * lax.rsqrt(var + 1e-5) * g2 + be2
    return


def _tc_fused(id2, fm, w1a, w1b, b1d, bd, b2d, mm, g2, be2):
    ntok = id2.shape[0] * 2
    grid = (ntok // BT,)
    full2 = lambda s: pl.BlockSpec(s, lambda i: (0, 0))
    return pl.pallas_call(
        _tc_body,
        grid=grid,
        in_specs=[
            pl.BlockSpec((BTH, 128), lambda i: (i, 0)),
            pl.BlockSpec((L, 128), lambda i: (i, 0)),
            full2((1, 128)), full2((1, 128)), full2((1, 128)),
            full2((128, 128)), full2((1, 128)), full2((128, 128)),
            full2((1, 128)), full2((1, 128)),
        ],
        out_specs=pl.BlockSpec((BTH, 128), lambda i: (i, 0)),
        out_shape=jax.ShapeDtypeStruct((ntok // 2, 128), jnp.float32),
    )(id2, fm, w1a, w1b, b1d, bd, b2d, mm, g2, be2)


NSEG = 2
RPS = B // (NSEG * NW)


@jax.jit
def kernel(item_seq, id_table, freq_table, W1, b1, W2, b2, gamma, beta):
    f0_tab = freq_table[:, 0].reshape(NROWS8, FW)
    f1_tab = freq_table[:, 1].reshape(NROWS8, FW)
    seg_rows = B // NSEG
    sc = _sc_gather_call(RPS)
    segs = []
    for s in range(NSEG):
        idx_s = item_seq[s * seg_rows:(s + 1) * seg_rows].reshape(NW, RPS, L)
        segs.append(sc(idx_s, id_table, f0_tab, f1_tab))

    w1a = jnp.tile(W1[0:1, :], (1, 2))
    w1b = jnp.tile(W1[1:2, :], (1, 2))
    b1d = jnp.tile(b1.reshape(1, H), (1, 2))
    b2d = jnp.tile(b2.reshape(1, H), (1, 2))
    g2 = jnp.tile(gamma.reshape(1, H), (1, 2))
    be2 = jnp.tile(beta.reshape(1, H), (1, 2))
    z64 = jnp.zeros((H, H), jnp.float32)
    bd = jnp.concatenate(
        [jnp.concatenate([W2, z64], axis=1),
         jnp.concatenate([z64, W2], axis=1)], axis=0)
    ii = jnp.arange(128)
    half = (ii < H)
    mm = jnp.where(half[:, None] == half[None, :], 1.0 / H, 0.0).astype(jnp.float32)

    outs = []
    for s in range(NSEG):
        id_emb, fm = segs[s]
        id2 = id_emb.reshape((N // NSEG) // 2, 128)
        o = _tc_fused(id2, fm, w1a, w1b, b1d, bd, b2d, mm, g2, be2)
        outs.append(o.reshape(seg_rows, L, H))
    return jnp.concatenate(outs, axis=0)


# speedup vs baseline: 1.0624x; 1.0624x over previous
import jax
import jax.numpy as jnp
from jax import lax
from jax.experimental import pallas as pl
from jax.experimental.pallas import tpu as pltpu
from jax.experimental.pallas import tpu_sc as plsc

B, L, H = 4096, 200, 64
N = B * L
NC, NS = 2, 16
NW = NC * NS
RPW = B // NW
CHUNK = L
FW = 8
NROWS8 = 1000000 // FW
PER_W = RPW * L


def _sc_gather_body(idx_hbm, id_tab, f0_tab, f1_tab, id_out, fm_out,
                    idx_v, idxc0, idx80, rowsa0, rowsb0, d00, d10, fmask0,
                    idxc1, idx81, rowsa1, rowsb1, d01, d11, fmask1,
                    semi0, semf0, semi1, semf1):
    wid = lax.axis_index("s") * NC + lax.axis_index("c")
    pltpu.sync_copy(idx_hbm.at[wid], idx_v)

    iota = lax.iota(jnp.int32, 16)
    tv_off = (iota >> 1) & 1
    evenl = (iota & 1) == 0
    low4 = iota < 4
    WINS = [0, 16, 32, 48, 64, 80, 96, 112, 128, 144, 160, 176, 184]
    bufs = ((idxc0, idx80, rowsa0, rowsb0, d00, d10, fmask0, semi0, semf0),
            (idxc1, idx81, rowsa1, rowsb1, d01, d11, fmask1, semi1, semf1))

    def fire(j, buf):
        idxc_v, idx8_v, rows_a, rows_b, d0_v, d1_v, _, sem_id, sem_fr = buf
        for off in WINS:
            v = idx_v[j, pl.ds(off, 16)]
            idxc_v[pl.ds(off, 16)] = v
            idx8_v[pl.ds(off, 16)] = v >> 3
        pltpu.async_copy(id_tab.at[idxc_v.at[pl.ds(0, 128)]], rows_a, sem_id)
        pltpu.async_copy(id_tab.at[idxc_v.at[pl.ds(128, 72)]], rows_b, sem_id)
        pltpu.async_copy(f0_tab.at[idx8_v.at[pl.ds(0, 128)]],
                         d0_v.at[pl.ds(0, 128)], sem_fr)
        pltpu.async_copy(f0_tab.at[idx8_v.at[pl.ds(128, 72)]],
                         d0_v.at[pl.ds(128, 72)], sem_fr)
        pltpu.async_copy(f1_tab.at[idx8_v.at[pl.ds(0, 128)]],
                         d1_v.at[pl.ds(0, 128)], sem_fr)
        pltpu.async_copy(f1_tab.at[idx8_v.at[pl.ds(128, 72)]],
                         d1_v.at[pl.ds(128, 72)], sem_fr)

    def consume(j, buf):
        idxc_v, idx8_v, rows_a, rows_b, d0_v, d1_v, fmask_v, sem_id, sem_fr = buf
        base = wid * PER_W + j * CHUNK
        pltpu.make_async_copy(id_tab.at[idxc_v.at[pl.ds(0, 128)]], rows_a,
                              sem_id).wait()
        pltpu.make_async_copy(id_tab.at[idxc_v.at[pl.ds(128, 72)]], rows_b,
                              sem_id).wait()
        pltpu.sync_copy(rows_a, id_out.at[pl.ds(base, 128)])
        pltpu.sync_copy(rows_b, id_out.at[pl.ds(base + 128, 72)])
        pltpu.make_async_copy(f0_tab.at[idx8_v.at[pl.ds(0, 128)]],
                              d0_v.at[pl.ds(0, 128)], sem_fr).wait()
        pltpu.make_async_copy(f0_tab.at[idx8_v.at[pl.ds(128, 72)]],
                              d0_v.at[pl.ds(128, 72)], sem_fr).wait()
        pltpu.make_async_copy(f1_tab.at[idx8_v.at[pl.ds(0, 128)]],
                              d1_v.at[pl.ds(0, 128)], sem_fr).wait()
        pltpu.make_async_copy(f1_tab.at[idx8_v.at[pl.ds(128, 72)]],
                              d1_v.at[pl.ds(128, 72)], sem_fr).wait()
        for i in range(100):
            tv = tv_off + 2 * i
            m = plsc.load_gather(idxc_v, [tv]) & 7
            a = plsc.load_gather(d0_v, [tv, m])
            bb = plsc.load_gather(d1_v, [tv, m])
            v = jnp.where(evenl, a, bb)
            fmask_v[i] = jnp.where(low4, v, 0.0)
        rr0 = base // 2
        row0 = 200 * (rr0 // 1600) + rr0 % 200
        km = (rr0 % 1600) // 200
        pltpu.sync_copy(fmask_v, fm_out.at[pl.ds(row0, 100), pl.ds(16 * km, 16)])

    fire(0, bufs[0])

    def body(jj, carry):
        j0 = 2 * jj
        fire(j0 + 1, bufs[1])
        consume(j0, bufs[0])

        @pl.when(jj < RPW // 2 - 1)
        def _():
            fire(j0 + 2, bufs[0])

        consume(j0 + 1, bufs[1])
        return carry

    lax.fori_loop(0, RPW // 2, body, 0)


_SC_CACHE = {}


def _sc_gather_call():
    if "k" not in _SC_CACHE:
        _SC_CACHE["k"] = pl.kernel(
            _sc_gather_body,
            out_type=(jax.ShapeDtypeStruct((N, H), jnp.float32),
                      jax.ShapeDtypeStruct((N // 16, 128), jnp.float32)),
            mesh=plsc.VectorSubcoreMesh(core_axis_name="c", subcore_axis_name="s"),
            compiler_params=pltpu.CompilerParams(use_tc_tiling_on_sc=False,
                                                 needs_layout_passes=False),
            scratch_types=[
                pltpu.VMEM((RPW, L), jnp.int32),
                pltpu.VMEM((L,), jnp.int32),
                pltpu.VMEM((L,), jnp.int32),
                pltpu.VMEM((128, H), jnp.float32),
                pltpu.VMEM((72, H), jnp.float32),
                pltpu.VMEM((L, FW), jnp.float32),
                pltpu.VMEM((L, FW), jnp.float32),
                pltpu.VMEM((100, 16), jnp.float32),
                pltpu.VMEM((L,), jnp.int32),
                pltpu.VMEM((L,), jnp.int32),
                pltpu.VMEM((128, H), jnp.float32),
                pltpu.VMEM((72, H), jnp.float32),
                pltpu.VMEM((L, FW), jnp.float32),
                pltpu.VMEM((L, FW), jnp.float32),
                pltpu.VMEM((100, 16), jnp.float32),
                pltpu.SemaphoreType.DMA,
                pltpu.SemaphoreType.DMA,
                pltpu.SemaphoreType.DMA,
                pltpu.SemaphoreType.DMA,
            ],
        )
    return _SC_CACHE["k"]


BT = 3200
BTH = BT // 2
BR = 16


def _tc_body(id_ref, fm_ref, w1a_ref, w1b_ref, b1_ref, bd_ref, b2_ref,
             m_ref, g_ref, be_ref, out_ref):
    fm = fm_ref[...]
    w1a = w1a_ref[...]
    w1b = w1b_ref[...]
    b1d = b1_ref[...]
    bd = bd_ref[...]
    b2d = b2_ref[...]
    mm = m_ref[...]
    g2 = g_ref[...]
    be2 = be_ref[...]
    lmask = lax.broadcasted_iota(jnp.int32, (1, 128), 1) < H
    for k in range(8):
        c = 16 * k
        F0 = jnp.where(lmask, fm[:, c + 0:c + 1], fm[:, c + 2:c + 3])
        F1 = jnp.where(lmask, fm[:, c + 1:c + 2], fm[:, c + 3:c + 4])
        h = jnp.tanh(F0 * w1a + F1 * w1b + b1d)
        fe = jnp.dot(h, bd, preferred_element_type=jnp.float32)
        x = id_ref[pl.ds(L * k, L), :] + fe + b2d
        mu = jnp.dot(x, mm, preferred_element_type=jnp.float32)
        d = x - mu
        var = jnp.dot(d * d, mm, preferred_element_type=jnp.float32)
        out_ref[pl.ds(L * k, L), :] = d * lax.rsqrt(var + 1e-5) * g2 + be2
    return


def _tc_fused(id2, fm, w1a, w1b, b1d, bd, b2d, mm, g2, be2):
    grid = (N // BT,)
    full2 = lambda s: pl.BlockSpec(s, lambda i: (0, 0))
    return pl.pallas_call(
        _tc_body,
        grid=grid,
        in_specs=[
            pl.BlockSpec((BTH, 128), lambda i: (i, 0)),
            pl.BlockSpec((L, 128), lambda i: (i, 0)),
            full2((1, 128)), full2((1, 128)), full2((1, 128)),
            full2((128, 128)), full2((1, 128)), full2((128, 128)),
            full2((1, 128)), full2((1, 128)),
        ],
        out_specs=pl.BlockSpec((BTH, 128), lambda i: (i, 0)),
        out_shape=jax.ShapeDtypeStruct((N // 2, 128), jnp.float32),
    )(id2, fm, w1a, w1b, b1d, bd, b2d, mm, g2, be2)


@jax.jit
def kernel(item_seq, id_table, freq_table, W1, b1, W2, b2, gamma, beta):
    idx = item_seq.reshape(NW, RPW, L)
    f0_tab = freq_table[:, 0].reshape(NROWS8, FW)
    f1_tab = freq_table[:, 1].reshape(NROWS8, FW)
    id_emb, fm = _sc_gather_call()(idx, id_table, f0_tab, f1_tab)
    id2 = id_emb.reshape(N // 2, 128)

    w1a = jnp.tile(W1[0:1, :], (1, 2))
    w1b = jnp.tile(W1[1:2, :], (1, 2))
    b1d = jnp.tile(b1.reshape(1, H), (1, 2))
    b2d = jnp.tile(b2.reshape(1, H), (1, 2))
    g2 = jnp.tile(gamma.reshape(1, H), (1, 2))
    be2 = jnp.tile(beta.reshape(1, H), (1, 2))
    z64 = jnp.zeros((H, H), jnp.float32)
    bd = jnp.concatenate(
        [jnp.concatenate([W2, z64], axis=1),
         jnp.concatenate([z64, W2], axis=1)], axis=0)
    ii = jnp.arange(128)
    half = (ii < H)
    mm = jnp.where(half[:, None] == half[None, :], 1.0 / H, 0.0).astype(jnp.float32)

    out = _tc_fused(id2, fm, w1a, w1b, b1d, bd, b2d, mm, g2, be2)
    return out.reshape(B, L, H)
